# Initial kernel scaffold; baseline (speedup 1.0000x reference)
import jax
import jax.numpy as jnp
from jax.experimental import pallas as pl


def kernel(img, x, W1, gamma, beta, W2, b2):
    raise NotImplementedError("write your pallas kernel here")


# speedup vs baseline: 2.0439x; 2.0439x over previous
import functools

import jax
import jax.numpy as jnp
import numpy as np
from jax import lax
from jax.experimental import pallas as pl
from jax.experimental.pallas import tpu as pltpu
from jax.experimental.pallas import tpu_sc as plsc

_H = 512
_P = 16
_G = _H // _P
_NP = _G * _G
_EPS_BN = 1e-5
_MASK_RATIO = 0.75

_kn = (np.outer(np.arange(_H), np.arange(_H)) % _H).astype(np.float64)
_ang = (2.0 * np.pi / _H) * _kn
_COS_NP = np.cos(_ang).astype(np.float32)
_SIN_NP = np.sin(_ang).astype(np.float32)
_POOL_NP = np.zeros((_G, _H), np.float32)
_POOL_NP[np.arange(_H) // _P, np.arange(_H)] = 1.0

_HIGH = jax.lax.Precision.HIGHEST


def _mm(a, b):
    return jnp.dot(a, b, precision=_HIGH, preferred_element_type=jnp.float32)


def _dft_body(img_ref, c_ref, s_ref, p_ref, pt_ref, m_ref, e_ref, st_ref):
    im = img_ref[0]
    c = c_ref[...]
    s = s_ref[...]
    t1r = _mm(c, im)
    t1i = -_mm(s, im)
    xr = _mm(t1r, c) + _mm(t1i, s)
    xi = _mm(t1i, c) - _mm(t1r, s)
    mag = jnp.sqrt(xr * xr + xi * xi) * (1.0 / _H)
    m_ref[0] = mag
    e_ref[0] = _mm(p_ref[...], _mm(mag, pt_ref[...]))
    s1 = jnp.sum(mag)
    s2 = jnp.sum(mag * mag)
    lane = lax.broadcasted_iota(jnp.int32, (1, 128), 1)
    st_ref[0] = jnp.where(lane == 0, s1, jnp.where(lane == 1, s2, 0.0))


def _bf16_rne(x):
    u = jax.lax.bitcast_convert_type(x, jnp.uint32)
    u = u + jnp.uint32(0x7FFF) + ((u >> 16) & jnp.uint32(1))
    return jax.lax.bitcast_convert_type(u & jnp.uint32(0xFFFF0000),
                                        jnp.float32)


def _fw_body(m_ref, e_ref, p_ref, pt_ref, par_ref, out_ref):
    mag = m_ref[0]
    acc = jnp.zeros((_H, _H), jnp.float32)
    for o in range(16):
        w1o = par_ref[o]
        muo = par_ref[16 + o]
        deno = par_ref[32 + o]
        go = par_ref[48 + o]
        bo = par_ref[64 + o]
        w2q = par_ref[80 + o]
        t = w1o * mag
        t = (t - muo) / deno
        t = t * go + bo
        t = jnp.maximum(t, 0.0)
        acc = acc + w2q * _bf16_rne(t)
    fw_sum = _mm(p_ref[...], _mm(acc, pt_ref[...]))
    fw_mean = fw_sum * (1.0 / (_P * _P)) + par_ref[96]
    sig = 1.0 / (1.0 + jnp.exp(-fw_mean))
    out_ref[0] = (e_ref[0] * (1.0 / (_P * _P))) * sig


def _rank_body(er_ref, ec_ref, nr_ref, nc_ref, mask_ref, idr_ref, idk_ref,
               len_keep: int, len_remove: int, L: int):
    e_r = er_ref[0]
    e_c = ec_ref[0]
    n_r = nr_ref[0]
    n_c = nc_ref[0]
    io = lax.broadcasted_iota(jnp.int32, (L, L), 0)
    jo = lax.broadcasted_iota(jnp.int32, (L, L), 1)
    gt = (e_r > e_c) | ((e_r == e_c) & (jo < io))
    rank_e_col = jnp.sum(gt.astype(jnp.int32), axis=1, keepdims=True)
    keep_col = rank_e_col < len_remove
    ltn = (n_c < n_r) | ((n_c == n_r) & (io < jo))
    rank_n_row = jnp.sum(ltn.astype(jnp.int32), axis=0, keepdims=True)
    idr_ref[0] = rank_n_row
    sel = (io == rank_n_row) & keep_col
    mask_ref[0] = jnp.sum(sel.astype(jnp.float32), axis=0, keepdims=True)
    ik = lax.broadcasted_iota(jnp.int32, (len_keep, L), 0)
    jk = lax.broadcasted_iota(jnp.int32, (len_keep, L), 1)
    hit = ik == rank_n_row
    idk_ref[0] = jnp.sum(jnp.where(hit, jk, 0), axis=1, keepdims=True)


@functools.lru_cache(maxsize=None)
def _build_sc_gather(V: int, D: int, B: int):
    info = plsc.get_sparse_core_info()
    nc, ns = info.num_cores, info.num_subcores
    nw = nc * ns
    b_per_w = B // nw
    chunk = 128
    nchunks = b_per_w // chunk
    mesh = plsc.VectorSubcoreMesh(core_axis_name="c", subcore_axis_name="s")

    @functools.partial(
        pl.kernel,
        mesh=mesh,
        out_type=jax.ShapeDtypeStruct((B, D), jnp.float32),
        scratch_types=[
            pltpu.VMEM((b_per_w,), jnp.int32),
            pltpu.VMEM((chunk, D), jnp.float32),
            pltpu.SemaphoreType.DMA,
        ],
    )
    def gather_kernel(table_hbm, idx_hbm, out_hbm, idx_v, rows_v, sem):
        wid = lax.axis_index("s") * nc + lax.axis_index("c")
        base = wid * b_per_w
        pltpu.sync_copy(idx_hbm.at[pl.ds(base, b_per_w)], idx_v)
        for ci in range(nchunks):
            pltpu.async_copy(
                table_hbm.at[idx_v.at[pl.ds(ci * chunk, chunk)]],
                rows_v, sem).wait()
            pltpu.sync_copy(rows_v, out_hbm.at[pl.ds(base + ci * chunk, chunk)])

    return gather_kernel


def _tc_pipeline(img, x, W1, gamma, beta, W2, b2):
    N, L, D = x.shape
    len_remove = int(L * _MASK_RATIO)
    len_keep = L - len_remove

    cmat = jnp.asarray(_COS_NP)
    smat = jnp.asarray(_SIN_NP)
    pool = jnp.asarray(_POOL_NP)
    poolT = jnp.asarray(_POOL_NP.T)

    img2 = img.reshape(N, _H, _H)

    mag, e_sums, stats = pl.pallas_call(
        _dft_body,
        grid=(N,),
        in_specs=[
            pl.BlockSpec((1, _H, _H), lambda n: (n, 0, 0)),
            pl.BlockSpec((_H, _H), lambda n: (0, 0)),
            pl.BlockSpec((_H, _H), lambda n: (0, 0)),
            pl.BlockSpec((_G, _H), lambda n: (0, 0)),
            pl.BlockSpec((_H, _G), lambda n: (0, 0)),
        ],
        out_specs=[
            pl.BlockSpec((1, _H, _H), lambda n: (n, 0, 0)),
            pl.BlockSpec((1, _G, _G), lambda n: (n, 0, 0)),
            pl.BlockSpec((1, 1, 128), lambda n: (n, 0, 0)),
        ],
        out_shape=[
            jax.ShapeDtypeStruct((N, _H, _H), jnp.float32),
            jax.ShapeDtypeStruct((N, _G, _G), jnp.float32),
            jax.ShapeDtypeStruct((N, 1, 128), jnp.float32),
        ],
    )(img2, cmat, smat, pool, poolT)

    tot = float(N * _H * _H)
    s1 = jnp.sum(stats[:, 0, 0])
    s2 = jnp.sum(stats[:, 0, 1])
    mean_m = s1 / tot
    var_m = s2 / tot - mean_m * mean_m
    w1 = W1.reshape(16)
    mu_ch = w1 * mean_m
    den_ch = jnp.sqrt(w1 * w1 * var_m + _EPS_BN)
    w2u = jax.lax.bitcast_convert_type(W2.reshape(16), jnp.uint32)
    w2u = w2u + jnp.uint32(0x7FFF) + ((w2u >> 16) & jnp.uint32(1))
    w2q = jax.lax.bitcast_convert_type(w2u & jnp.uint32(0xFFFF0000),
                                       jnp.float32)
    params = jnp.concatenate(
        [w1, mu_ch, den_ch, gamma, beta, w2q, b2.reshape(1),
         jnp.zeros((31,), jnp.float32)])

    energy = pl.pallas_call(
        _fw_body,
        grid=(N,),
        in_specs=[
            pl.BlockSpec((1, _H, _H), lambda n: (n, 0, 0)),
            pl.BlockSpec((1, _G, _G), lambda n: (n, 0, 0)),
            pl.BlockSpec((_G, _H), lambda n: (0, 0)),
            pl.BlockSpec((_H, _G), lambda n: (0, 0)),
            pl.BlockSpec(memory_space=pltpu.SMEM),
        ],
        out_specs=pl.BlockSpec((1, _G, _G), lambda n: (n, 0, 0)),
        out_shape=jax.ShapeDtypeStruct((N, _G, _G), jnp.float32),
    )(mag, e_sums, pool, poolT, params)

    noise = jax.random.uniform(jax.random.key(42), (N, L), dtype=jnp.float32)
    e_row = energy.reshape(N, 1, L)
    e_col = energy.reshape(N, L, 1)
    n_row = noise.reshape(N, 1, L)
    n_col = noise.reshape(N, L, 1)

    mask3, idr3, idk3 = pl.pallas_call(
        functools.partial(_rank_body, len_keep=len_keep,
                          len_remove=len_remove, L=L),
        grid=(N,),
        in_specs=[
            pl.BlockSpec((1, 1, L), lambda n: (n, 0, 0)),
            pl.BlockSpec((1, L, 1), lambda n: (n, 0, 0)),
            pl.BlockSpec((1, 1, L), lambda n: (n, 0, 0)),
            pl.BlockSpec((1, L, 1), lambda n: (n, 0, 0)),
        ],
        out_specs=[
            pl.BlockSpec((1, 1, L), lambda n: (n, 0, 0)),
            pl.BlockSpec((1, 1, L), lambda n: (n, 0, 0)),
            pl.BlockSpec((1, len_keep, 1), lambda n: (n, 0, 0)),
        ],
        out_shape=[
            jax.ShapeDtypeStruct((N, 1, L), jnp.float32),
            jax.ShapeDtypeStruct((N, 1, L), jnp.int32),
            jax.ShapeDtypeStruct((N, len_keep, 1), jnp.int32),
        ],
    )(e_row, e_col, n_row, n_col)

    mask = mask3.reshape(N, L)
    ids_restore = idr3.reshape(N, L)
    ids_keep = idk3.reshape(N, len_keep)
    return mask, ids_restore, ids_keep


def kernel(img, x, W1, gamma, beta, W2, b2):
    N, L, D = x.shape
    mask, ids_restore, ids_keep = _tc_pipeline(
        img, x, W1, gamma, beta, W2, b2)
    len_keep = ids_keep.shape[1]
    flat_idx = ids_keep + jnp.arange(N, dtype=jnp.int32)[:, None] * L
    x_flat = x.reshape(N * L, D)
    gather = _build_sc_gather(N * L, D, N * len_keep)
    x_masked = gather(x_flat, flat_idx.reshape(-1)).reshape(N, len_keep, D)
    return (x_masked, mask, ids_restore, ids_keep)


# speedup vs baseline: 3.0006x; 1.4681x over previous
import functools

import jax
import jax.numpy as jnp
import numpy as np
from jax import lax
from jax.experimental import pallas as pl
from jax.experimental.pallas import tpu as pltpu
from jax.experimental.pallas import tpu_sc as plsc

_H = 512
_P = 16
_G = _H // _P
_NP = _G * _G
_EPS_BN = 1e-5
_MASK_RATIO = 0.75

_kn = (np.outer(np.arange(_H), np.arange(_H)) % _H).astype(np.float64)
_ang = (2.0 * np.pi / _H) * _kn
_COS_NP = np.cos(_ang).astype(np.float32)
_SIN_NP = np.sin(_ang).astype(np.float32)
_POOL_NP = np.zeros((_G, _H), np.float32)
_POOL_NP[np.arange(_H) // _P, np.arange(_H)] = 1.0
_P1_NP = np.zeros((_G, 264), np.float32)
_P1_NP[:, :_H // 2 + 1] = _POOL_NP[:, :_H // 2 + 1]
_RM_NP = np.zeros((255, 264), np.float32)
_RM_NP[np.arange(255), 255 - np.arange(255)] = 1.0
_P2R_NP = _POOL_NP[:, _H // 2 + 1:] @ _RM_NP
_QC_NP = np.zeros((_H, _H), np.float32)
_QC_NP[(_H - np.arange(_H)) % _H, np.arange(_H)] = 1.0
_QPT_NP = _QC_NP @ _POOL_NP.T

_HIGH = jax.lax.Precision.HIGHEST


def _mm(a, b):
    return jnp.dot(a, b, precision=_HIGH, preferred_element_type=jnp.float32)


def _pool_sym(mpad, p1_ref, p2r_ref, pt_ref, qpt_ref):
    return (_mm(p1_ref[...], _mm(mpad, pt_ref[...]))
            + _mm(p2r_ref[...], _mm(mpad, qpt_ref[...])))


def _sym_sum(x):
    return (2.0 * jnp.sum(x) - jnp.sum(x[0:1, :]) - jnp.sum(x[256:257, :]))


def _dft_body(img_ref, c_ref, s_ref, p1_ref, p2r_ref, pt_ref, qpt_ref,
              m_ref, e_ref, st_ref):
    im = img_ref[0]
    c = c_ref[...]
    s = s_ref[...]
    t1r = _mm(c[0:257, :], im)
    t1i = -_mm(s[0:257, :], im)
    xr = _mm(t1r, c) + _mm(t1i, s)
    xi = _mm(t1i, c) - _mm(t1r, s)
    mtop = jnp.sqrt(xr * xr + xi * xi) * (1.0 / _H)
    mpad = jnp.concatenate(
        [mtop, jnp.zeros((7, _H), jnp.float32)], axis=0)
    m_ref[0] = mpad
    e_ref[0] = _pool_sym(mpad, p1_ref, p2r_ref, pt_ref, qpt_ref)
    s1 = _sym_sum(mpad)
    s2 = _sym_sum(mpad * mpad)
    lane = lax.broadcasted_iota(jnp.int32, (1, 128), 1)
    st_ref[0] = jnp.where(lane == 0, s1, jnp.where(lane == 1, s2, 0.0))


def _bf16_rne(x):
    u = jax.lax.bitcast_convert_type(x, jnp.uint32)
    u = u + jnp.uint32(0x7FFF) + ((u >> 16) & jnp.uint32(1))
    return jax.lax.bitcast_convert_type(u & jnp.uint32(0xFFFF0000),
                                        jnp.float32)


def _fw_body(m_ref, e_ref, p1_ref, p2r_ref, pt_ref, qpt_ref, par_ref,
             out_ref):
    mag = m_ref[0]
    acc = jnp.zeros((264, _H), jnp.float32)
    for o in range(16):
        w1o = par_ref[o]
        muo = par_ref[16 + o]
        deno = par_ref[32 + o]
        go = par_ref[48 + o]
        bo = par_ref[64 + o]
        w2q = par_ref[80 + o]
        t = w1o * mag
        t = (t - muo) / deno
        t = t * go + bo
        t = jnp.maximum(t, 0.0)
        acc = acc + w2q * _bf16_rne(t)
    fw_sum = _pool_sym(acc, p1_ref, p2r_ref, pt_ref, qpt_ref)
    fw_mean = fw_sum * (1.0 / (_P * _P)) + par_ref[96]
    sig = 1.0 / (1.0 + jnp.exp(-fw_mean))
    out_ref[0] = (e_ref[0] * (1.0 / (_P * _P))) * sig


def _rank_body(er_ref, ec_ref, nr_ref, nc_ref, mask_ref, idr_ref, idk_ref,
               len_keep: int, len_remove: int, L: int):
    e_r = er_ref[0]
    e_c = ec_ref[0]
    n_r = nr_ref[0]
    n_c = nc_ref[0]
    io = lax.broadcasted_iota(jnp.int32, (L, L), 0)
    jo = lax.broadcasted_iota(jnp.int32, (L, L), 1)
    gt = (e_r > e_c) | ((e_r == e_c) & (jo < io))
    rank_e_col = jnp.sum(gt.astype(jnp.int32), axis=1, keepdims=True)
    keep_col = rank_e_col < len_remove
    ltn = (n_c < n_r) | ((n_c == n_r) & (io < jo))
    rank_n_row = jnp.sum(ltn.astype(jnp.int32), axis=0, keepdims=True)
    idr_ref[0] = rank_n_row
    sel = (io == rank_n_row) & keep_col
    mask_ref[0] = jnp.sum(sel.astype(jnp.float32), axis=0, keepdims=True)
    ik = lax.broadcasted_iota(jnp.int32, (len_keep, L), 0)
    jk = lax.broadcasted_iota(jnp.int32, (len_keep, L), 1)
    hit = ik == rank_n_row
    idk_ref[0] = jnp.sum(jnp.where(hit, jk, 0), axis=1, keepdims=True)


@functools.lru_cache(maxsize=None)
def _build_sc_gather(V: int, D: int, B: int):
    info = plsc.get_sparse_core_info()
    nc, ns = info.num_cores, info.num_subcores
    nw = nc * ns
    b_per_w = B // nw
    chunk = 128
    nchunks = b_per_w // chunk
    mesh = plsc.VectorSubcoreMesh(core_axis_name="c", subcore_axis_name="s")

    @functools.partial(
        pl.kernel,
        mesh=mesh,
        out_type=jax.ShapeDtypeStruct((B, D), jnp.float32),
        scratch_types=[
            pltpu.VMEM((b_per_w,), jnp.int32),
            pltpu.VMEM((chunk, D), jnp.float32),
            pltpu.SemaphoreType.DMA,
        ],
    )
    def gather_kernel(table_hbm, idx_hbm, out_hbm, idx_v, rows_v, sem):
        wid = lax.axis_index("s") * nc + lax.axis_index("c")
        base = wid * b_per_w
        pltpu.sync_copy(idx_hbm.at[pl.ds(base, b_per_w)], idx_v)
        for ci in range(nchunks):
            pltpu.async_copy(
                table_hbm.at[idx_v.at[pl.ds(ci * chunk, chunk)]],
                rows_v, sem).wait()
            pltpu.sync_copy(rows_v, out_hbm.at[pl.ds(base + ci * chunk, chunk)])

    return gather_kernel


def _tc_pipeline(img, x, W1, gamma, beta, W2, b2):
    N, L, D = x.shape
    len_remove = int(L * _MASK_RATIO)
    len_keep = L - len_remove

    cmat = jnp.asarray(_COS_NP)
    smat = jnp.asarray(_SIN_NP)
    p1 = jnp.asarray(_P1_NP)
    p2r = jnp.asarray(_P2R_NP)
    poolT = jnp.asarray(_POOL_NP.T)
    qpt = jnp.asarray(_QPT_NP)

    img2 = img.reshape(N, _H, _H)

    mag, e_sums, stats = pl.pallas_call(
        _dft_body,
        grid=(N,),
        in_specs=[
            pl.BlockSpec((1, _H, _H), lambda n: (n, 0, 0)),
            pl.BlockSpec((_H, _H), lambda n: (0, 0)),
            pl.BlockSpec((_H, _H), lambda n: (0, 0)),
            pl.BlockSpec((_G, 264), lambda n: (0, 0)),
            pl.BlockSpec((_G, 264), lambda n: (0, 0)),
            pl.BlockSpec((_H, _G), lambda n: (0, 0)),
            pl.BlockSpec((_H, _G), lambda n: (0, 0)),
        ],
        out_specs=[
            pl.BlockSpec((1, 264, _H), lambda n: (n, 0, 0)),
            pl.BlockSpec((1, _G, _G), lambda n: (n, 0, 0)),
            pl.BlockSpec((1, 1, 128), lambda n: (n, 0, 0)),
        ],
        out_shape=[
            jax.ShapeDtypeStruct((N, 264, _H), jnp.float32),
            jax.ShapeDtypeStruct((N, _G, _G), jnp.float32),
            jax.ShapeDtypeStruct((N, 1, 128), jnp.float32),
        ],
    )(img2, cmat, smat, p1, p2r, poolT, qpt)

    tot = float(N * _H * _H)
    s1 = jnp.sum(stats[:, 0, 0])
    s2 = jnp.sum(stats[:, 0, 1])
    mean_m = s1 / tot
    var_m = s2 / tot - mean_m * mean_m
    w1 = W1.reshape(16)
    mu_ch = w1 * mean_m
    den_ch = jnp.sqrt(w1 * w1 * var_m + _EPS_BN)
    w2u = jax.lax.bitcast_convert_type(W2.reshape(16), jnp.uint32)
    w2u = w2u + jnp.uint32(0x7FFF) + ((w2u >> 16) & jnp.uint32(1))
    w2q = jax.lax.bitcast_convert_type(w2u & jnp.uint32(0xFFFF0000),
                                       jnp.float32)
    params = jnp.concatenate(
        [w1, mu_ch, den_ch, gamma, beta, w2q, b2.reshape(1),
         jnp.zeros((31,), jnp.float32)])

    energy = pl.pallas_call(
        _fw_body,
        grid=(N,),
        in_specs=[
            pl.BlockSpec((1, 264, _H), lambda n: (n, 0, 0)),
            pl.BlockSpec((1, _G, _G), lambda n: (n, 0, 0)),
            pl.BlockSpec((_G, 264), lambda n: (0, 0)),
            pl.BlockSpec((_G, 264), lambda n: (0, 0)),
            pl.BlockSpec((_H, _G), lambda n: (0, 0)),
            pl.BlockSpec((_H, _G), lambda n: (0, 0)),
            pl.BlockSpec(memory_space=pltpu.SMEM),
        ],
        out_specs=pl.BlockSpec((1, _G, _G), lambda n: (n, 0, 0)),
        out_shape=jax.ShapeDtypeStruct((N, _G, _G), jnp.float32),
    )(mag, e_sums, p1, p2r, poolT, qpt, params)

    noise = jax.random.uniform(jax.random.key(42), (N, L), dtype=jnp.float32)
    e_row = energy.reshape(N, 1, L)
    e_col = energy.reshape(N, L, 1)
    n_row = noise.reshape(N, 1, L)
    n_col = noise.reshape(N, L, 1)

    mask3, idr3, idk3 = pl.pallas_call(
        functools.partial(_rank_body, len_keep=len_keep,
                          len_remove=len_remove, L=L),
        grid=(N,),
        in_specs=[
            pl.BlockSpec((1, 1, L), lambda n: (n, 0, 0)),
            pl.BlockSpec((1, L, 1), lambda n: (n, 0, 0)),
            pl.BlockSpec((1, 1, L), lambda n: (n, 0, 0)),
            pl.BlockSpec((1, L, 1), lambda n: (n, 0, 0)),
        ],
        out_specs=[
            pl.BlockSpec((1, 1, L), lambda n: (n, 0, 0)),
            pl.BlockSpec((1, 1, L), lambda n: (n, 0, 0)),
            pl.BlockSpec((1, len_keep, 1), lambda n: (n, 0, 0)),
        ],
        out_shape=[
            jax.ShapeDtypeStruct((N, 1, L), jnp.float32),
            jax.ShapeDtypeStruct((N, 1, L), jnp.int32),
            jax.ShapeDtypeStruct((N, len_keep, 1), jnp.int32),
        ],
    )(e_row, e_col, n_row, n_col)

    mask = mask3.reshape(N, L)
    ids_restore = idr3.reshape(N, L)
    ids_keep = idk3.reshape(N, len_keep)
    return mask, ids_restore, ids_keep


def kernel(img, x, W1, gamma, beta, W2, b2):
    N, L, D = x.shape
    mask, ids_restore, ids_keep = _tc_pipeline(
        img, x, W1, gamma, beta, W2, b2)
    len_keep = ids_keep.shape[1]
    flat_idx = ids_keep + jnp.arange(N, dtype=jnp.int32)[:, None] * L
    x_flat = x.reshape(N * L, D)
    gather = _build_sc_gather(N * L, D, N * len_keep)
    x_masked = gather(x_flat, flat_idx.reshape(-1)).reshape(N, len_keep, D)
    return (x_masked, mask, ids_restore, ids_keep)
